# baseline (speedup 1.0000x reference)
import functools

import jax
import jax.numpy as jnp
from jax import lax
from jax.experimental import pallas as pl
from jax.experimental.pallas import tpu as pltpu
from jax.experimental.pallas import tpu_sc as plsc

N = 16384
D = 64
L = 16
NC = 2
NS = 16
NW = NC * NS
R = N // NW
CH = 256
NCHUNK = R // CH

_mesh = plsc.VectorSubcoreMesh(core_axis_name="c", subcore_axis_name="s")


@functools.partial(
    pl.kernel,
    out_type=jax.ShapeDtypeStruct((N,), jnp.float32),
    mesh=_mesh,
    compiler_params=pltpu.CompilerParams(
        needs_layout_passes=False,
        disable_bounds_checks=True,
        disable_semaphore_checks=True,
        skip_device_barrier=True,
    ),
    scratch_types=[
        pltpu.VMEM((2, D, CH), jnp.float32),
        pltpu.VMEM((R,), jnp.float32),
        pltpu.VMEM((1, D), jnp.float32),
        pltpu.SemaphoreType.DMA,
        pltpu.SemaphoreType.DMA,
    ],
)
def _sc_matvec(user_hbm, items_t_hbm, out_hbm, buf, out_v, u_v, sem_in, sem_u):
    wid = lax.axis_index("s") * NC + lax.axis_index("c")
    base = wid * R

    ucp = pltpu.async_copy(user_hbm, u_v, sem_u)
    copies = [
        pltpu.async_copy(items_t_hbm.at[:, pl.ds(base + c * CH, CH)],
                         buf.at[c % 2], sem_in)
        for c in range(min(2, NCHUNK))
    ]
    ucp.wait()
    uregs = [u_v[0, pl.ds(k * L, L)] for k in range(D // L)]
    us = [uregs[c // L][c % L] for c in range(D)]

    for ch in range(NCHUNK):
        copies[ch].wait()
        cur = ch % 2

        def group_body(g, _, cur=cur, off=ch * CH):
            acc = us[0] * buf[cur, 0, pl.ds(g * L, L)]
            for c in range(1, D):
                acc = acc + us[c] * buf[cur, c, pl.ds(g * L, L)]
            out_v[pl.ds(off + g * L, L)] = acc
            return 0

        lax.fori_loop(0, CH // L, group_body, 0)

        if ch + 2 < NCHUNK:
            copies.append(
                pltpu.async_copy(items_t_hbm.at[:, pl.ds(base + (ch + 2) * CH, CH)],
                                 buf.at[ch % 2], sem_in))

    pltpu.sync_copy(out_v, out_hbm.at[pl.ds(base, R)])


def kernel(user_emb, items_emb):
    return _sc_matvec(user_emb, items_emb.T)


# speedup vs baseline: 1.0013x; 1.0013x over previous
import functools

import jax
import jax.numpy as jnp
from jax import lax
from jax.experimental import pallas as pl
from jax.experimental.pallas import tpu as pltpu
from jax.experimental.pallas import tpu_sc as plsc

N = 16384
D = 64
L = 16
NC = 2
NS = 16
NW = NC * NS
R = N // NW
CH = 256
NCHUNK = R // CH

_mesh = plsc.VectorSubcoreMesh(core_axis_name="c", subcore_axis_name="s")


@functools.partial(
    pl.kernel,
    out_type=jax.ShapeDtypeStruct((N,), jnp.float32),
    mesh=_mesh,
    compiler_params=pltpu.CompilerParams(needs_layout_passes=False),
    scratch_types=[
        pltpu.VMEM((2, D, CH), jnp.float32),
        pltpu.VMEM((R,), jnp.float32),
        pltpu.VMEM((1, D), jnp.float32),
        pltpu.SemaphoreType.DMA,
        pltpu.SemaphoreType.DMA,
    ],
)
def _sc_matvec(user_hbm, items_t_hbm, out_hbm, buf, out_v, u_v, sem_in, sem_u):
    wid = lax.axis_index("s") * NC + lax.axis_index("c")
    base = wid * R

    ucp = pltpu.async_copy(user_hbm, u_v, sem_u)
    copies = [
        pltpu.async_copy(items_t_hbm.at[:, pl.ds(base + c * CH, CH)],
                         buf.at[c % 2], sem_in)
        for c in range(min(2, NCHUNK))
    ]
    ucp.wait()
    uregs = [u_v[0, pl.ds(k * L, L)] for k in range(D // L)]
    G = 4

    for ch in range(NCHUNK):
        copies[ch].wait()
        cur = ch % 2

        def group_body(g, _, cur=cur, off=ch * CH):
            acc = [None] * G
            for c in range(D):
                bc = jnp.full((L,), uregs[c // L][c % L])
                for j in range(G):
                    t = bc * buf[cur, c, pl.ds((g * G + j) * L, L)]
                    acc[j] = t if acc[j] is None else acc[j] + t
            for j in range(G):
                out_v[pl.ds(off + (g * G + j) * L, L)] = acc[j]
            return 0

        lax.fori_loop(0, CH // (G * L), group_body, 0)

        if ch + 2 < NCHUNK:
            copies.append(
                pltpu.async_copy(items_t_hbm.at[:, pl.ds(base + (ch + 2) * CH, CH)],
                                 buf.at[ch % 2], sem_in))

    pltpu.sync_copy(out_v, out_hbm.at[pl.ds(base, R)])


def kernel(user_emb, items_emb):
    return _sc_matvec(user_emb, items_emb.T)


# speedup vs baseline: 1.0290x; 1.0277x over previous
import functools

import jax
import jax.numpy as jnp
from jax import lax
from jax.experimental import pallas as pl
from jax.experimental.pallas import tpu as pltpu
from jax.experimental.pallas import tpu_sc as plsc

N = 16384
D = 64
L = 16
NC = 2
NS = 16
NW = NC * NS
R = N // NW
CH = 512
NCHUNK = R // CH

_mesh = plsc.VectorSubcoreMesh(core_axis_name="c", subcore_axis_name="s")


@functools.partial(
    pl.kernel,
    out_type=jax.ShapeDtypeStruct((N,), jnp.float32),
    mesh=_mesh,
    compiler_params=pltpu.CompilerParams(needs_layout_passes=False),
    scratch_types=[
        pltpu.VMEM((2, D, CH), jnp.float32),
        pltpu.VMEM((R,), jnp.float32),
        pltpu.VMEM((1, D), jnp.float32),
        pltpu.SemaphoreType.DMA,
        pltpu.SemaphoreType.DMA,
    ],
)
def _sc_matvec(user_hbm, items_t_hbm, out_hbm, buf, out_v, u_v, sem_in, sem_u):
    wid = lax.axis_index("s") * NC + lax.axis_index("c")
    base = wid * R

    ucp = pltpu.async_copy(user_hbm, u_v, sem_u)
    copies = [
        pltpu.async_copy(items_t_hbm.at[:, pl.ds(base + c * CH, CH)],
                         buf.at[c % 2], sem_in)
        for c in range(min(2, NCHUNK))
    ]
    ucp.wait()
    uregs = [u_v[0, pl.ds(k * L, L)] for k in range(D // L)]
    G = 4

    for ch in range(NCHUNK):
        copies[ch].wait()
        cur = ch % 2

        def group_body(g, _, cur=cur, off=ch * CH):
            acc = [None] * G
            for c in range(D):
                bc = jnp.full((L,), uregs[c // L][c % L])
                for j in range(G):
                    t = bc * buf[cur, c, pl.ds((g * G + j) * L, L)]
                    acc[j] = t if acc[j] is None else acc[j] + t
            for j in range(G):
                out_v[pl.ds(off + (g * G + j) * L, L)] = acc[j]
            return 0

        lax.fori_loop(0, CH // (G * L), group_body, 0)

        if ch + 2 < NCHUNK:
            copies.append(
                pltpu.async_copy(items_t_hbm.at[:, pl.ds(base + (ch + 2) * CH, CH)],
                                 buf.at[ch % 2], sem_in))

    pltpu.sync_copy(out_v, out_hbm.at[pl.ds(base, R)])


def kernel(user_emb, items_emb):
    return _sc_matvec(user_emb, items_emb.T)


# speedup vs baseline: 1.1197x; 1.0882x over previous
import functools

import jax
import jax.numpy as jnp
from jax import lax
from jax.experimental import pallas as pl
from jax.experimental.pallas import tpu as pltpu
from jax.experimental.pallas import tpu_sc as plsc

N = 16384
D = 64
L = 16
NC = 2
NS = 16
NW = NC * NS
R = N // NW
NG = R // L

_mesh = plsc.VectorSubcoreMesh(core_axis_name="c", subcore_axis_name="s")


@functools.partial(
    pl.kernel,
    out_type=jax.ShapeDtypeStruct((N,), jnp.float32),
    mesh=_mesh,
    compiler_params=pltpu.CompilerParams(needs_layout_passes=False),
    scratch_types=[
        pltpu.VMEM((D, R), jnp.float32),
        pltpu.VMEM((R,), jnp.float32),
        pltpu.VMEM((1, D), jnp.float32),
        pltpu.SemaphoreType.DMA,
        pltpu.SemaphoreType.DMA,
    ],
)
def _sc_matvec(user_hbm, items_t_hbm, out_hbm, buf, out_v, u_v, sem_in, sem_u):
    wid = lax.axis_index("s") * NC + lax.axis_index("c")
    base = wid * R

    ucp = pltpu.async_copy(user_hbm, u_v, sem_u)
    icp = pltpu.async_copy(items_t_hbm.at[:, pl.ds(base, R)], buf, sem_in)
    ucp.wait()
    icp.wait()

    dnums = lax.GatherDimensionNumbers(
        offset_dims=(), collapsed_slice_dims=(0,), start_index_map=(0,))
    zero = lax.broadcast(jnp.float32(0), (L,))

    def col_body(c, accs):
        c16 = (c // L) * L
        uvec = u_v[0, pl.ds(c16, L)]
        idx = lax.broadcast(c - c16, (L,))
        bc = lax.gather(uvec, idx.reshape(L, 1), dnums, (1,),
                        mode=lax.GatherScatterMode.PROMISE_IN_BOUNDS)
        return tuple(accs[g] + bc * buf[c, pl.ds(g * L, L)]
                     for g in range(NG))

    accs = lax.fori_loop(0, D, col_body, (zero,) * NG)
    for g in range(NG):
        out_v[pl.ds(g * L, L)] = accs[g]

    pltpu.sync_copy(out_v, out_hbm.at[pl.ds(base, R)])


def kernel(user_emb, items_emb):
    return _sc_matvec(user_emb, items_emb.T)
